# baseline (speedup 1.0000x reference)
import functools

import jax
import jax.numpy as jnp
from jax import lax
from jax.experimental import pallas as pl
from jax.experimental.pallas import tpu as pltpu
from jax.experimental.pallas import tpu_sc as plsc

N = 100000
K = 64
B = 1024

NC = 2
NS = 16
L = 16
NW = NC * NS
BPW = B // NW
RPW = 2 * K // NW

_mesh = plsc.VectorSubcoreMesh(core_axis_name="c", subcore_axis_name="s")


@functools.partial(
    pl.kernel,
    out_type=[
        jax.ShapeDtypeStruct((K, B), jnp.float32),
        jax.ShapeDtypeStruct((K, B), jnp.float32),
        jax.ShapeDtypeStruct((B,), jnp.float32),
        jax.ShapeDtypeStruct((B,), jnp.float32),
    ],
    mesh=_mesh,
    compiler_params=pltpu.CompilerParams(needs_layout_passes=False),
    scratch_types=[
        pltpu.VMEM((B,), jnp.int32),
        pltpu.VMEM((N,), jnp.float32),
        pltpu.VMEM((B,), jnp.float32),
        pltpu.VMEM((B,), jnp.float32),
        pltpu.VMEM((BPW,), jnp.float32),
        pltpu.VMEM((BPW,), jnp.float32),
        pltpu.SemaphoreType.DMA,
        pltpu.SemaphoreType.DMA,
    ],
)
def _sc_encoder(idx_hbm, wpm_hbm, wpl_hbm, whm_hbm, whl_hbm,
                pm_hbm, pv_hbm, hm_hbm, hv_hbm,
                idx_v, row_v, exta_v, extb_v, hm_v, hv_v, sem, semw):
    wid = lax.axis_index("s") * NC + lax.axis_index("c")
    base_b = wid * BPW

    pltpu.sync_copy(idx_hbm, idx_v)

    hcps = [
        pltpu.async_copy(whm_hbm.at[0].at[idx_v.at[pl.ds(base_b, BPW)]],
                         hm_v, sem),
        pltpu.async_copy(whl_hbm.at[0].at[idx_v.at[pl.ds(base_b, BPW)]],
                         hv_v, sem),
    ]

    wcps = []
    for t in range(RPW):
        src = (wpm_hbm, wpm_hbm, wpl_hbm, wpl_hbm)[t]
        dst = (pm_hbm, pm_hbm, pv_hbm, pv_hbm)[t]
        ext_v = (exta_v, extb_v)[t % 2]
        k = wid + (t % 2) * NW
        pltpu.sync_copy(src.at[k], row_v)

        def extract(j, carry, apply_exp=(t >= 2), ext_v=ext_v):
            base = pl.multiple_of(j * L, L)
            iv = idx_v[pl.ds(base, L)]
            x = plsc.load_gather(row_v, [iv])
            if apply_exp:
                x = jnp.exp(x * 2.0)
            ext_v[pl.ds(base, L)] = x
            return carry

        lax.fori_loop(0, B // L, extract, 0, unroll=8)
        if len(wcps) >= 2:
            wcps[t - 2].wait()
        wcps.append(pltpu.async_copy(ext_v, dst.at[k], semw))

    for cp in hcps:
        cp.wait()
    for i in range(BPW // L):
        hv_v[pl.ds(i * L, L)] = jnp.exp(hv_v[pl.ds(i * L, L)] * 2.0)
    wcps.append(pltpu.async_copy(hm_v, hm_hbm.at[pl.ds(base_b, BPW)], semw))
    wcps.append(pltpu.async_copy(hv_v, hv_hbm.at[pl.ds(base_b, BPW)], semw))
    for cp in wcps[-4:]:
        cp.wait()


def kernel(indices, W_pos_mean, W_pos_logvar, W_het_mean, W_het_logvar):
    idx = indices.astype(jnp.int32)
    pm_t, pv_t, hm, hv = _sc_encoder(
        idx, W_pos_mean, W_pos_logvar, W_het_mean, W_het_logvar,
    )
    return (
        pm_t.T,
        pv_t.T,
        hm.reshape(B, 1),
        hv.reshape(B, 1),
    )


# speedup vs baseline: 1.0038x; 1.0038x over previous
import functools

import jax
import jax.numpy as jnp
from jax import lax
from jax.experimental import pallas as pl
from jax.experimental.pallas import tpu as pltpu
from jax.experimental.pallas import tpu_sc as plsc

N = 100000
K = 64
B = 1024

NC = 2
NS = 16
L = 16
NW = NC * NS
BPW = B // NW
RPW = 2 * K // NW

_mesh = plsc.VectorSubcoreMesh(core_axis_name="c", subcore_axis_name="s")


@functools.partial(
    pl.kernel,
    out_type=[
        jax.ShapeDtypeStruct((K, B), jnp.float32),
        jax.ShapeDtypeStruct((K, B), jnp.float32),
        jax.ShapeDtypeStruct((B,), jnp.float32),
        jax.ShapeDtypeStruct((B,), jnp.float32),
    ],
    mesh=_mesh,
    compiler_params=pltpu.CompilerParams(needs_layout_passes=False),
    scratch_types=[
        pltpu.VMEM((B,), jnp.int32),
        pltpu.VMEM((N,), jnp.float32),
        pltpu.VMEM((B,), jnp.float32),
        pltpu.VMEM((B,), jnp.float32),
        pltpu.VMEM((BPW,), jnp.float32),
        pltpu.VMEM((BPW,), jnp.float32),
        pltpu.SemaphoreType.DMA,
        pltpu.SemaphoreType.DMA,
    ],
)
def _sc_encoder(idx_hbm, wpm_hbm, wpl_hbm, whm_hbm, whl_hbm,
                pm_hbm, pv_hbm, hm_hbm, hv_hbm,
                idx_v, row_v, exta_v, extb_v, hm_v, hv_v, sem, semw):
    wid = lax.axis_index("s") * NC + lax.axis_index("c")
    base_b = wid * BPW

    pltpu.sync_copy(idx_hbm, idx_v)

    hcps = [
        pltpu.async_copy(whm_hbm.at[0].at[idx_v.at[pl.ds(base_b, BPW)]],
                         hm_v, sem),
        pltpu.async_copy(whl_hbm.at[0].at[idx_v.at[pl.ds(base_b, BPW)]],
                         hv_v, sem),
    ]

    wcps = []
    for t in range(RPW):
        src = (wpm_hbm, wpm_hbm, wpl_hbm, wpl_hbm)[t]
        dst = (pm_hbm, pm_hbm, pv_hbm, pv_hbm)[t]
        ext_v = (exta_v, extb_v)[t % 2]
        k = wid + (t % 2) * NW
        pltpu.sync_copy(src.at[k], row_v)

        def extract(j, carry, apply_exp=(t >= 2), ext_v=ext_v):
            base = pl.multiple_of(j * L, L)
            iv = idx_v[pl.ds(base, L)]
            x = plsc.load_gather(row_v, [iv])
            if apply_exp:
                x = jnp.exp(x * 2.0)
            ext_v[pl.ds(base, L)] = x
            return carry

        lax.fori_loop(0, B // L, extract, 0, unroll=4)
        if len(wcps) >= 2:
            wcps[t - 2].wait()
        wcps.append(pltpu.async_copy(ext_v, dst.at[k], semw))

    for cp in hcps:
        cp.wait()
    for i in range(BPW // L):
        hv_v[pl.ds(i * L, L)] = jnp.exp(hv_v[pl.ds(i * L, L)] * 2.0)
    pltpu.sync_copy(hm_v, hm_hbm.at[pl.ds(base_b, BPW)])
    pltpu.sync_copy(hv_v, hv_hbm.at[pl.ds(base_b, BPW)])
    for cp in wcps[-2:]:
        cp.wait()


def kernel(indices, W_pos_mean, W_pos_logvar, W_het_mean, W_het_logvar):
    idx = indices.astype(jnp.int32)
    pm_t, pv_t, hm, hv = _sc_encoder(
        idx, W_pos_mean, W_pos_logvar, W_het_mean, W_het_logvar,
    )
    return (
        pm_t.T,
        pv_t.T,
        hm.reshape(B, 1),
        hv.reshape(B, 1),
    )
